# baseline (speedup 1.0000x reference)
import functools

import jax
import jax.numpy as jnp
from jax import lax
from jax.experimental import pallas as pl
from jax.experimental.pallas import tpu as pltpu
from jax.experimental.pallas import tpu_sc as plsc

_B = 512
_N = 16
_L = 16
_NC = 2
_NS = 16

_GATHER_DNUMS = lax.GatherDimensionNumbers(
    offset_dims=(), collapsed_slice_dims=(0,), start_index_map=(0,))


def _permute(v, idx):
    return lax.gather(v, idx[:, None], _GATHER_DNUMS, (1,),
                      mode=lax.GatherScatterMode.PROMISE_IN_BOUNDS)


def _butterfly_transpose(vecs):
    lane = lax.iota(jnp.int32, _L)
    for s in (1, 2, 4, 8):
        perm = lane ^ s
        cond = [jnp.asarray((jnp.arange(_L) & s) == (i & s)) for i in range(_L)]
        vecs = [
            jnp.where(cond[i], vecs[i], _permute(vecs[i ^ s], perm))
            for i in range(_L)
        ]
    return vecs


def _build_sc_call():
    mesh = plsc.VectorSubcoreMesh(core_axis_name="c", subcore_axis_name="s")

    @functools.partial(
        pl.kernel,
        mesh=mesh,
        out_type=jax.ShapeDtypeStruct((_B, _N * _N), jnp.float32),
        scratch_types=[
            pltpu.VMEM((_L, _N * _N), jnp.float32),
            pltpu.VMEM((_L, _N * _N), jnp.float32),
        ],
    )
    def sc_stick_breaking(x_hbm, out_hbm, x_v, out_v):
        wid = lax.axis_index("s") * _NC + lax.axis_index("c")
        base = wid * _L
        pltpu.sync_copy(x_hbm.at[pl.ds(base, _L)], x_v)

        zero = jnp.zeros((_L,), jnp.float32)
        one = jnp.ones((_L,), jnp.float32)

        def one_row(m, col_sums):
            rows = [x_v[j, pl.ds(m * _N, _N)] for j in range(_L)]
            xb = _butterfly_transpose(rows)
            suffix = [zero] * _N
            acc = zero
            for n in range(_N - 1, 0, -1):
                acc = acc + col_sums[n]
                suffix[n - 1] = acc
            sum_row = zero
            new_cols = list(col_sums)
            outs = []
            for n in range(_N):
                bv = one / (one + jnp.exp(-xb[n]))
                fl = one - bv
                an = jnp.full((_L,), float(2 - _N + n), jnp.float32) + suffix[n]
                vn = one - new_cols[n]
                lower = jnp.maximum(zero, an - sum_row)
                upper = jnp.minimum(one - sum_row, vn)
                p = fl * lower + bv * upper
                outs.append(p)
                sum_row = sum_row + p
                new_cols[n] = new_cols[n] + p
            obk = _butterfly_transpose(outs)
            for j in range(_L):
                out_v[j, pl.ds(m * _N, _N)] = obk[j]
            return tuple(new_cols)

        def row_body(h, col_sums):
            return one_row(h * 2 + 1, one_row(h * 2, col_sums))

        lax.fori_loop(0, _N // 2, row_body, tuple([zero] * _N))
        pltpu.sync_copy(out_v, out_hbm.at[pl.ds(base, _L)])

    return sc_stick_breaking


_SC_CALL = _build_sc_call()


def kernel(x):
    out = _SC_CALL(x.reshape(_B, _N * _N))
    return out.reshape(_B, _N, _N)


# speedup vs baseline: 1.0230x; 1.0230x over previous
import functools

import jax
import jax.numpy as jnp
from jax import lax
from jax.experimental import pallas as pl
from jax.experimental.pallas import tpu as pltpu
from jax.experimental.pallas import tpu_sc as plsc

_B = 512
_N = 16
_L = 16
_NC = 2
_NS = 16

_GATHER_DNUMS = lax.GatherDimensionNumbers(
    offset_dims=(), collapsed_slice_dims=(0,), start_index_map=(0,))


def _permute(v, idx):
    return lax.gather(v, idx[:, None], _GATHER_DNUMS, (1,),
                      mode=lax.GatherScatterMode.PROMISE_IN_BOUNDS)


def _butterfly_transpose(vecs):
    lane = lax.iota(jnp.int32, _L)
    for s in (1, 2, 4, 8):
        perm = lane ^ s
        cond = [jnp.asarray((jnp.arange(_L) & s) == (i & s)) for i in range(_L)]
        vecs = [
            jnp.where(cond[i], vecs[i], _permute(vecs[i ^ s], perm))
            for i in range(_L)
        ]
    return vecs


def _build_sc_call():
    mesh = plsc.VectorSubcoreMesh(core_axis_name="c", subcore_axis_name="s")

    @functools.partial(
        pl.kernel,
        mesh=mesh,
        out_type=jax.ShapeDtypeStruct((_B, _N * _N), jnp.float32),
        scratch_types=[
            pltpu.VMEM((_L, _N * _N), jnp.float32),
            pltpu.VMEM((_L, _N * _N), jnp.float32),
        ],
    )
    def sc_stick_breaking(x_hbm, out_hbm, x_v, out_v):
        wid = lax.axis_index("s") * _NC + lax.axis_index("c")
        base = wid * _L
        pltpu.sync_copy(x_hbm.at[pl.ds(base, _L)], x_v)

        zero = jnp.zeros((_L,), jnp.float32)
        one = jnp.ones((_L,), jnp.float32)

        def row_body(m, col_sums):
            rows = [x_v[j, pl.ds(m * _N, _N)] for j in range(_L)]
            xb = _butterfly_transpose(rows)
            suffix = [zero] * _N
            acc = zero
            for n in range(_N - 1, 0, -1):
                acc = acc + col_sums[n]
                suffix[n - 1] = acc
            sum_row = zero
            new_cols = list(col_sums)
            outs = []
            for n in range(_N):
                bv = one / (one + jnp.exp(-xb[n]))
                fl = one - bv
                an = jnp.full((_L,), float(2 - _N + n), jnp.float32) + suffix[n]
                vn = one - new_cols[n]
                lower = jnp.maximum(zero, an - sum_row)
                upper = jnp.minimum(one - sum_row, vn)
                p = fl * lower + bv * upper
                outs.append(p)
                sum_row = sum_row + p
                new_cols[n] = new_cols[n] + p
            obk = _butterfly_transpose(outs)
            for j in range(_L):
                out_v[j, pl.ds(m * _N, _N)] = obk[j]
            return tuple(new_cols)

        lax.fori_loop(0, _N, row_body, tuple([zero] * _N))
        pltpu.sync_copy(out_v, out_hbm.at[pl.ds(base, _L)])

    return sc_stick_breaking


_SC_CALL = _build_sc_call()


def kernel(x):
    out = _SC_CALL(x.reshape(_B, _N * _N))
    return out.reshape(_B, _N, _N)
